# baseline (speedup 1.0000x reference)
import functools

import numpy as np
import jax
import jax.numpy as jnp
from jax import lax
from jax.experimental import pallas as pl
from jax.experimental.pallas import tpu as pltpu
from jax.experimental.pallas import tpu_sc as plsc

N = 10000
F_IN = 128
F_OUT = 32
F3 = 3 * F_OUT

NC = 2
NS = 16
NW = NC * NS

ROW = 128
RCH = 1
CE = ROW * RCH
DCH = 16
FW = F3 // 32
NPT = 640
SUB = 80

_SC_PARAMS = pltpu.CompilerParams(
    use_tc_tiling_on_sc=False, needs_layout_passes=False)


def _rsqrt_newton(d):
    i = plsc.bitcast(d, jnp.int32)
    y = plsc.bitcast(jnp.int32(0x5F3759DF) - (i >> 1), jnp.float32)
    for _ in range(3):
        y = y * (1.5 - 0.5 * d * y * y)
    return y


def _unpack_bf16_pair(bf_vreg):
    word = plsc.bitcast(bf_vreg, jnp.int32)
    lo = plsc.bitcast(word << 16, jnp.float32)
    hi = plsc.bitcast(word & jnp.int32(-65536), jnp.float32)
    return lo, hi


def _sc_degree(dst2d, w2d, n_rows):
    rows_per_core = n_rows // NC
    rows_per_tile = rows_per_core // NS
    n_chunks = rows_per_tile // DCH

    mesh = plsc.VectorSubcoreMesh(core_axis_name="c", subcore_axis_name="s")

    @functools.partial(
        pl.kernel,
        out_type=jax.ShapeDtypeStruct((NC, N), jnp.float32),
        mesh=mesh,
        scratch_types=[
            pltpu.VMEM((2, DCH, ROW), jnp.int32),
            pltpu.VMEM((2, DCH, ROW), jnp.float32),
            pltpu.VMEM((1000,), jnp.float32),
            pltpu.VMEM_SHARED((N,), jnp.float32),
            pltpu.SemaphoreType.DMA,
            pltpu.SemaphoreType.DMA,
        ],
        compiler_params=_SC_PARAMS,
    )
    def kern(dst_hbm, w_hbm, degp_out, dstb, wb, zl, deg_sh, sem_e, sem_s):
        c = lax.axis_index("c")
        s = lax.axis_index("s")
        z16 = jnp.zeros((16,), jnp.float32)

        @pl.loop(0, 1000 // 16)
        def _(i):
            zl[pl.ds(i * 16, 16)] = z16

        @pl.when(s < 10)
        def _():
            pltpu.sync_copy(zl, deg_sh.at[pl.ds(s * 1000, 1000)])

        plsc.subcore_barrier()

        base = c * rows_per_core + s * rows_per_tile

        def edge_dma(g, b, start):
            f = pltpu.async_copy if start else (
                lambda *a: pltpu.make_async_copy(*a).wait())
            f(dst_hbm.at[pl.ds(base + g * DCH, DCH)], dstb.at[b], sem_e)
            f(w_hbm.at[pl.ds(base + g * DCH, DCH)], wb.at[b], sem_e)

        edge_dma(0, 0, True)
        for g in range(n_chunks):
            b = g % 2
            edge_dma(g, b, False)
            if g >= 1:
                for j in range(DCH):
                    pltpu.make_async_copy(
                        wb.at[1 - b].at[j],
                        deg_sh.at[dstb.at[1 - b].at[j]], sem_s).wait()
            if g + 1 < n_chunks:
                edge_dma(g + 1, 1 - b, True)
            for j in range(DCH):
                pltpu.async_copy(wb.at[b].at[j],
                                 deg_sh.at[dstb.at[b].at[j]], sem_s,
                                 add=True)
        bl = (n_chunks - 1) % 2
        for j in range(DCH):
            pltpu.make_async_copy(wb.at[bl].at[j],
                                  deg_sh.at[dstb.at[bl].at[j]], sem_s).wait()

        plsc.subcore_barrier()

        @pl.when(s < 10)
        def _():
            sl = pl.ds(s * 1000, 1000)
            pltpu.sync_copy(deg_sh.at[sl], degp_out.at[c].at[sl])

    return kern(dst2d, w2d)


def _sc_messages(src2d, dst2d, w2d, hb, degp, n_rows):
    pair_rows = n_rows // NS
    rw0 = int(round(pair_rows * 0.58 / (2 * RCH))) * 2 * RCH
    rw0 = min(max(rw0, 2 * RCH), pair_rows - 2 * RCH)
    rw1 = pair_rows - rw0
    nch0, nch1 = rw0 // RCH, rw1 // RCH

    mesh = plsc.VectorSubcoreMesh(core_axis_name="c", subcore_axis_name="s")

    @functools.partial(
        pl.kernel,
        out_type=jax.ShapeDtypeStruct((NC, N, F3), jnp.float32),
        mesh=mesh,
        scratch_types=[
            pltpu.VMEM((N,), jnp.float32),
            pltpu.VMEM((2000,), jnp.float32),
            pltpu.VMEM((2, RCH, ROW), jnp.int32),
            pltpu.VMEM((2, RCH, ROW), jnp.int32),
            pltpu.VMEM((2, RCH, ROW), jnp.float32),
            pltpu.VMEM((2, CE, F3), jnp.bfloat16),
            pltpu.VMEM((2, CE, F3), jnp.float32),
            pltpu.VMEM_SHARED((N, F3), jnp.float32),
            pltpu.SemaphoreType.DMA,
            pltpu.SemaphoreType.DMA,
            pltpu.SemaphoreType.DMA,
            pltpu.SemaphoreType.DMA,
        ],
        compiler_params=_SC_PARAMS,
    )
    def kern(src_hbm, dst_hbm, w_hbm, hb_hbm, degp_hbm, acc_out,
             dinv_l, tp, srcb, dstb, wb, rowsb, rowsf, acc_sh,
             sem_e, sem_g, sem_sa, sem_sb):
        c = lax.axis_index("c")
        s = lax.axis_index("s")
        sem_s = (sem_sa, sem_sb)
        z16 = jnp.zeros((16,), jnp.float32)

        for t in range(N // 2000):
            sl = pl.ds(t * 2000, 2000)
            pltpu.sync_copy(degp_hbm.at[0].at[sl], dinv_l.at[sl])
            pltpu.sync_copy(degp_hbm.at[1].at[sl], tp)

            @pl.loop(0, 2000 // 16)
            def _(i):
                si = pl.ds(i * 16, 16)
                so = pl.ds(t * 2000 + i * 16, 16)
                dinv_l[so] = _rsqrt_newton(dinv_l[so] + tp[si] + 1.0)

        node0 = s * NPT
        cnt_subs = jnp.where(s < NS - 1, NPT // SUB, (N - (NS - 1) * NPT) // SUB)

        @pl.when(c == 0)
        def _():
            @pl.loop(0, cnt_subs)
            def _(q):
                n0 = node0 + q * SUB
                pltpu.sync_copy(hb_hbm.at[pl.ds(n0, SUB)],
                                rowsb.at[0].at[pl.ds(0, SUB)])

                @pl.loop(0, SUB // 16)
                def _(g):
                    dvv = dinv_l[pl.ds(n0 + g * 16, 16)]
                    for j in range(16):
                        a = dvv[j]
                        e = g * 16 + j
                        for f in range(FW):
                            wv = rowsb[0, e, pl.ds(f * 32, 32)]
                            lo, hi = _unpack_bf16_pair(wv)
                            rowsf[0, e, pl.ds(f * 32, 16)] = lo * a
                            rowsf[0, e, pl.ds(f * 32 + 16, 16)] = hi * a

                pltpu.sync_copy(rowsf.at[0].at[pl.ds(0, SUB)],
                                acc_sh.at[pl.ds(n0, SUB)])

        @pl.when(c == 1)
        def _():
            @pl.loop(0, CE)
            def _(e):
                for f in range(F3 // 16):
                    rowsf[0, e, pl.ds(f * 16, 16)] = z16

            @pl.loop(0, cnt_subs)
            def _(q):
                pltpu.sync_copy(rowsf.at[0].at[pl.ds(0, SUB)],
                                acc_sh.at[pl.ds(node0 + q * SUB, SUB)])

        plsc.subcore_barrier()

        n_chunks = jnp.where(c == 0, nch0, nch1)
        w0row = jnp.where(c == 0, s * rw0, NS * rw0 + s * rw1)

        def edge_dma(k, b, start):
            f = pltpu.async_copy if start else (
                lambda *a: pltpu.make_async_copy(*a).wait())
            r0 = pl.ds(w0row + k * RCH, RCH)
            f(src_hbm.at[r0], srcb.at[b], sem_e)
            f(dst_hbm.at[r0], dstb.at[b], sem_e)
            f(w_hbm.at[r0], wb.at[b], sem_e)

        def gather(b, start):
            for j in range(RCH):
                srcr = hb_hbm.at[srcb.at[b].at[j]]
                dstr = rowsb.at[b].at[pl.ds(j * ROW, ROW)]
                if start:
                    pltpu.async_copy(srcr, dstr, sem_g)
                else:
                    pltpu.make_async_copy(srcr, dstr, sem_g).wait()

        def scatter(b, start):
            sem = sem_s[b]
            for j in range(RCH):
                srcr = rowsf.at[b].at[pl.ds(j * ROW, ROW)]
                dstr = acc_sh.at[dstb.at[b].at[j]]
                if start:
                    pltpu.async_copy(srcr, dstr, sem, add=True)
                else:
                    pltpu.make_async_copy(srcr, dstr, sem).wait()

        edge_dma(0, 0, True)
        edge_dma(0, 0, False)
        gather(0, True)

        @pl.loop(0, (nch0 + 1) // 2)
        def _(g):
            for b in range(2):
                k = g * 2 + b

                @pl.when(k < n_chunks)
                def _():
                    @pl.when(k >= 1)
                    def _():
                        scatter(1 - b, False)

                    @pl.when(k + 1 < n_chunks)
                    def _():
                        edge_dma(k + 1, 1 - b, True)

                    gather(b, False)

                    @pl.when(k + 1 < n_chunks)
                    def _():
                        edge_dma(k + 1, 1 - b, False)
                        gather(1 - b, True)

                    @pl.loop(0, RCH)
                    def _(r):
                        @pl.loop(0, ROW // 16)
                        def _(i):
                            idx = srcb[b, r, pl.ds(i * 16, 16)]
                            dv = plsc.load_gather(dinv_l, [idx])
                            av = wb[b, r, pl.ds(i * 16, 16)] * dv
                            e0 = r * ROW + i * 16
                            for j in range(16):
                                a = av[j]
                                e = e0 + j
                                for f in range(FW):
                                    wv = rowsb[b, e, pl.ds(f * 32, 32)]
                                    lo, hi = _unpack_bf16_pair(wv)
                                    rowsf[b, e, pl.ds(f * 32, 16)] = lo * a
                                    rowsf[b, e, pl.ds(f * 32 + 16, 16)] = \
                                        hi * a

                    scatter(b, True)

        scatter(1, False)
        plsc.subcore_barrier()

        @pl.loop(0, cnt_subs)
        def _(q):
            n0 = node0 + q * SUB
            pltpu.sync_copy(acc_sh.at[pl.ds(n0, SUB)],
                            rowsf.at[0].at[pl.ds(0, SUB)])

            @pl.loop(0, SUB // 16)
            def _(g):
                dvv = dinv_l[pl.ds(n0 + g * 16, 16)]
                for j in range(16):
                    a = dvv[j]
                    e = g * 16 + j
                    for f in range(F3 // 16):
                        sf = pl.ds(f * 16, 16)
                        rowsf[0, e, sf] = rowsf[0, e, sf] * a

            pltpu.sync_copy(rowsf.at[0].at[pl.ds(0, SUB)],
                            acc_out.at[c].at[pl.ds(n0, SUB)])

    return kern(src2d, dst2d, w2d, hb, degp)


def _tc_project(x, wcat_p):
    nb = 5
    bs = N // nb

    def body(x_ref, w_ref, o_ref):
        o_ref[...] = jnp.dot(x_ref[...], w_ref[...],
                             preferred_element_type=jnp.float32
                             ).astype(jnp.bfloat16)

    return pl.pallas_call(
        body,
        grid=(nb,),
        in_specs=[
            pl.BlockSpec((bs, F_IN), lambda i: (i, 0)),
            pl.BlockSpec((F_IN, F3), lambda i: (0, 0)),
        ],
        out_specs=pl.BlockSpec((bs, F3), lambda i: (i, 0)),
        out_shape=jax.ShapeDtypeStruct((N, F3), jnp.bfloat16),
    )(x, wcat_p)


def _tc_gru(acc, hprev, wza, wzb, cz, wra, wrb, cr, wha, whb, ch,
            wlin, blin):
    nb = 5
    bs = N // nb

    def body(a0_ref, a1_ref, hp_ref, wza_ref, wzb_ref, cz_ref,
             wra_ref, wrb_ref, cr_ref, wha_ref, whb_ref, ch_ref,
             wlin_ref, blin_ref, y_ref, hn_ref):
        hp = hp_ref[...]
        agg = a0_ref[0] + a1_ref[0]
        gz = agg[:, :F_OUT]
        gr = agg[:, F_OUT:2 * F_OUT]
        gh = agg[:, 2 * F_OUT:]
        f32 = jnp.float32
        z = jax.nn.sigmoid(jnp.dot(gz, wza_ref[...], preferred_element_type=f32)
                           + jnp.dot(hp, wzb_ref[...], preferred_element_type=f32)
                           + cz_ref[...])
        r = jax.nn.sigmoid(jnp.dot(gr, wra_ref[...], preferred_element_type=f32)
                           + jnp.dot(hp, wrb_ref[...], preferred_element_type=f32)
                           + cr_ref[...])
        ht = jnp.tanh(jnp.dot(gh, wha_ref[...], preferred_element_type=f32)
                      + jnp.dot(hp * r, whb_ref[...], preferred_element_type=f32)
                      + ch_ref[...])
        hn = z * hp + (1.0 - z) * ht
        hn_ref[...] = hn
        y_ref[...] = (jnp.dot(jax.nn.relu(hn), wlin_ref[...],
                              preferred_element_type=f32) + blin_ref[...])

    full = lambda r, c: pl.BlockSpec((r, c), lambda i: (0, 0))
    blk = lambda cdim: pl.BlockSpec((bs, cdim), lambda i: (i, 0))
    acc_blk = lambda ci: pl.BlockSpec((1, bs, F3), lambda i, ci=ci: (ci, i, 0))
    return pl.pallas_call(
        body,
        grid=(nb,),
        in_specs=[
            acc_blk(0), acc_blk(1), blk(F_OUT),
            full(F_OUT, F_OUT), full(F_OUT, F_OUT), full(1, F_OUT),
            full(F_OUT, F_OUT), full(F_OUT, F_OUT), full(1, F_OUT),
            full(F_OUT, F_OUT), full(F_OUT, F_OUT), full(1, F_OUT),
            full(F_OUT, 1), full(1, 1),
        ],
        out_specs=[blk(1), blk(F_OUT)],
        out_shape=[
            jax.ShapeDtypeStruct((N, 1), jnp.float32),
            jax.ShapeDtypeStruct((N, F_OUT), jnp.float32),
        ],
    )(acc, acc, hprev, wza, wzb, cz, wra, wrb, cr,
      wha, whb, ch, wlin, blin)


_PERM = np.empty((F3,), dtype=np.int32)
for _t in range(FW):
    for _j in range(16):
        _PERM[32 * _t + 2 * _j] = 32 * _t + _j
        _PERM[32 * _t + 2 * _j + 1] = 32 * _t + 16 + _j


def kernel(x, edge_index, edge_weight, prev_hidden_state, c,
           Wz_c, bz_c, Wr_c, br_c, Wh_c, bh_c,
           Wz, bz, Wr, br, Wh, bh, Wlin, blin):
    src, dst = edge_index[0], edge_index[1]
    e = src.shape[0]

    grain = NC * NS * DCH * ROW
    epad = -(-e // grain) * grain
    pad = epad - e
    if pad:
        src = jnp.concatenate([src, jnp.zeros((pad,), src.dtype)])
        dst = jnp.concatenate([dst, jnp.zeros((pad,), dst.dtype)])
        edge_weight = jnp.concatenate(
            [edge_weight, jnp.zeros((pad,), edge_weight.dtype)])
    n_rows = epad // ROW
    src2d = src.reshape(n_rows, ROW)
    dst2d = dst.reshape(n_rows, ROW)
    w2d = edge_weight.reshape(n_rows, ROW)

    wcat = jnp.concatenate([Wz_c, Wr_c, Wh_c], axis=1)
    wcat_p = wcat[:, _PERM]
    cz = (bz_c @ Wz[:F_OUT] + bz).reshape(1, F_OUT)
    cr = (br_c @ Wr[:F_OUT] + br).reshape(1, F_OUT)
    ch = (bh_c @ Wh[:F_OUT] + bh).reshape(1, F_OUT)

    degp = _sc_degree(dst2d, w2d, n_rows)
    hb = _tc_project(x, wcat_p)
    acc = _sc_messages(src2d, dst2d, w2d, hb, degp, n_rows)
    y, hn = _tc_gru(acc, prev_hidden_state,
                    Wz[:F_OUT], Wz[F_OUT:], cz,
                    Wr[:F_OUT], Wr[F_OUT:], cr,
                    Wh[:F_OUT], Wh[F_OUT:], ch,
                    Wlin, blin.reshape(1, 1))
    return (y, hn)


# speedup vs baseline: 1.0440x; 1.0440x over previous
import functools

import numpy as np
import jax
import jax.numpy as jnp
from jax import lax
from jax.experimental import pallas as pl
from jax.experimental.pallas import tpu as pltpu
from jax.experimental.pallas import tpu_sc as plsc

N = 10000
F_IN = 128
F_OUT = 32
F3 = 3 * F_OUT

NC = 2
NS = 16
NW = NC * NS

ROW = 128
RCH = 1
CE = ROW * RCH
DCH = 16
FW = F3 // 32
NPT = 640
SUB = 80

_SC_PARAMS = pltpu.CompilerParams(
    use_tc_tiling_on_sc=False, needs_layout_passes=False)


def _rsqrt_newton(d):
    i = plsc.bitcast(d, jnp.int32)
    y = plsc.bitcast(jnp.int32(0x5F3759DF) - (i >> 1), jnp.float32)
    for _ in range(3):
        y = y * (1.5 - 0.5 * d * y * y)
    return y


def _unpack_bf16_pair(bf_vreg):
    word = plsc.bitcast(bf_vreg, jnp.int32)
    lo = plsc.bitcast(word << 16, jnp.float32)
    hi = plsc.bitcast(word & jnp.int32(-65536), jnp.float32)
    return lo, hi


def _sc_degree(dst2d, w2d, n_rows):
    rows_per_core = n_rows // NC
    rows_per_tile = rows_per_core // NS
    n_chunks = rows_per_tile // DCH

    mesh = plsc.VectorSubcoreMesh(core_axis_name="c", subcore_axis_name="s")

    @functools.partial(
        pl.kernel,
        out_type=jax.ShapeDtypeStruct((NC, N), jnp.float32),
        mesh=mesh,
        scratch_types=[
            pltpu.VMEM((2, DCH, ROW), jnp.int32),
            pltpu.VMEM((2, DCH, ROW), jnp.float32),
            pltpu.VMEM((1000,), jnp.float32),
            pltpu.VMEM_SHARED((N,), jnp.float32),
            pltpu.SemaphoreType.DMA,
            pltpu.SemaphoreType.DMA,
        ],
        compiler_params=_SC_PARAMS,
    )
    def kern(dst_hbm, w_hbm, degp_out, dstb, wb, zl, deg_sh, sem_e, sem_s):
        c = lax.axis_index("c")
        s = lax.axis_index("s")
        z16 = jnp.zeros((16,), jnp.float32)

        @pl.loop(0, 1000 // 16)
        def _(i):
            zl[pl.ds(i * 16, 16)] = z16

        @pl.when(s < 10)
        def _():
            pltpu.sync_copy(zl, deg_sh.at[pl.ds(s * 1000, 1000)])

        plsc.subcore_barrier()

        base = c * rows_per_core + s * rows_per_tile

        def edge_dma(g, b, start):
            f = pltpu.async_copy if start else (
                lambda *a: pltpu.make_async_copy(*a).wait())
            f(dst_hbm.at[pl.ds(base + g * DCH, DCH)], dstb.at[b], sem_e)
            f(w_hbm.at[pl.ds(base + g * DCH, DCH)], wb.at[b], sem_e)

        edge_dma(0, 0, True)
        for g in range(n_chunks):
            b = g % 2
            edge_dma(g, b, False)
            if g >= 1:
                for j in range(DCH):
                    pltpu.make_async_copy(
                        wb.at[1 - b].at[j],
                        deg_sh.at[dstb.at[1 - b].at[j]], sem_s).wait()
            if g + 1 < n_chunks:
                edge_dma(g + 1, 1 - b, True)
            for j in range(DCH):
                pltpu.async_copy(wb.at[b].at[j],
                                 deg_sh.at[dstb.at[b].at[j]], sem_s,
                                 add=True)
        bl = (n_chunks - 1) % 2
        for j in range(DCH):
            pltpu.make_async_copy(wb.at[bl].at[j],
                                  deg_sh.at[dstb.at[bl].at[j]], sem_s).wait()

        plsc.subcore_barrier()

        @pl.when(s < 10)
        def _():
            sl = pl.ds(s * 1000, 1000)
            pltpu.sync_copy(deg_sh.at[sl], degp_out.at[c].at[sl])

    return kern(dst2d, w2d)


def _sc_messages(src2d, dst2d, w2d, hb, degp, n_rows):
    pair_rows = n_rows // NS
    rw0 = int(round(pair_rows * 0.54 / (2 * RCH))) * 2 * RCH
    rw0 = min(max(rw0, 2 * RCH), pair_rows - 2 * RCH)
    rw1 = pair_rows - rw0
    nch0, nch1 = rw0 // RCH, rw1 // RCH

    mesh = plsc.VectorSubcoreMesh(core_axis_name="c", subcore_axis_name="s")

    @functools.partial(
        pl.kernel,
        out_type=jax.ShapeDtypeStruct((NC, N, F3), jnp.float32),
        mesh=mesh,
        scratch_types=[
            pltpu.VMEM((N,), jnp.float32),
            pltpu.VMEM((2000,), jnp.float32),
            pltpu.VMEM((2, RCH, ROW), jnp.int32),
            pltpu.VMEM((2, RCH, ROW), jnp.int32),
            pltpu.VMEM((2, RCH, ROW), jnp.float32),
            pltpu.VMEM((2, CE, F3), jnp.bfloat16),
            pltpu.VMEM((2, CE, F3), jnp.float32),
            pltpu.VMEM((16,), jnp.float32),
            pltpu.VMEM_SHARED((N, F3), jnp.float32),
            pltpu.SemaphoreType.DMA,
            pltpu.SemaphoreType.DMA,
            pltpu.SemaphoreType.DMA,
            pltpu.SemaphoreType.DMA,
        ],
        compiler_params=_SC_PARAMS,
    )
    def kern(src_hbm, dst_hbm, w_hbm, hb_hbm, degp_hbm, acc_out,
             dinv_l, tp, srcb, dstb, wb, rowsb, rowsf, a16, acc_sh,
             sem_e, sem_g, sem_sa, sem_sb):
        c = lax.axis_index("c")
        s = lax.axis_index("s")
        sem_s = (sem_sa, sem_sb)
        z16 = jnp.zeros((16,), jnp.float32)

        for t in range(N // 2000):
            sl = pl.ds(t * 2000, 2000)
            pltpu.sync_copy(degp_hbm.at[0].at[sl], dinv_l.at[sl])
            pltpu.sync_copy(degp_hbm.at[1].at[sl], tp)

            @pl.loop(0, 2000 // 16)
            def _(i):
                si = pl.ds(i * 16, 16)
                so = pl.ds(t * 2000 + i * 16, 16)
                dinv_l[so] = _rsqrt_newton(dinv_l[so] + tp[si] + 1.0)

        node0 = s * NPT
        cnt_subs = jnp.where(s < NS - 1, NPT // SUB, (N - (NS - 1) * NPT) // SUB)

        @pl.when(c == 0)
        def _():
            @pl.loop(0, cnt_subs)
            def _(q):
                n0 = node0 + q * SUB
                pltpu.sync_copy(hb_hbm.at[pl.ds(n0, SUB)],
                                rowsb.at[0].at[pl.ds(0, SUB)])

                @pl.loop(0, SUB // 16)
                def _(g):
                    dvv = dinv_l[pl.ds(n0 + g * 16, 16)]
                    for j in range(16):
                        a = dvv[j]
                        e = g * 16 + j
                        for f in range(FW):
                            wv = rowsb[0, e, pl.ds(f * 32, 32)]
                            lo, hi = _unpack_bf16_pair(wv)
                            rowsf[0, e, pl.ds(f * 32, 16)] = lo * a
                            rowsf[0, e, pl.ds(f * 32 + 16, 16)] = hi * a

                pltpu.sync_copy(rowsf.at[0].at[pl.ds(0, SUB)],
                                acc_sh.at[pl.ds(n0, SUB)])

        @pl.when(c == 1)
        def _():
            @pl.loop(0, CE)
            def _(e):
                for f in range(F3 // 16):
                    rowsf[0, e, pl.ds(f * 16, 16)] = z16

            @pl.loop(0, cnt_subs)
            def _(q):
                pltpu.sync_copy(rowsf.at[0].at[pl.ds(0, SUB)],
                                acc_sh.at[pl.ds(node0 + q * SUB, SUB)])

        plsc.subcore_barrier()

        n_chunks = jnp.where(c == 0, nch0, nch1)
        w0row = jnp.where(c == 0, s * rw0, NS * rw0 + s * rw1)

        def edge_dma(k, b, start):
            f = pltpu.async_copy if start else (
                lambda *a: pltpu.make_async_copy(*a).wait())
            r0 = pl.ds(w0row + k * RCH, RCH)
            f(src_hbm.at[r0], srcb.at[b], sem_e)
            f(dst_hbm.at[r0], dstb.at[b], sem_e)
            f(w_hbm.at[r0], wb.at[b], sem_e)

        def gather(b, start):
            for j in range(RCH):
                srcr = hb_hbm.at[srcb.at[b].at[j]]
                dstr = rowsb.at[b].at[pl.ds(j * ROW, ROW)]
                if start:
                    pltpu.async_copy(srcr, dstr, sem_g)
                else:
                    pltpu.make_async_copy(srcr, dstr, sem_g).wait()

        def scatter(b, start):
            sem = sem_s[b]
            for j in range(RCH):
                srcr = rowsf.at[b].at[pl.ds(j * ROW, ROW)]
                dstr = acc_sh.at[dstb.at[b].at[j]]
                if start:
                    pltpu.async_copy(srcr, dstr, sem, add=True)
                else:
                    pltpu.make_async_copy(srcr, dstr, sem).wait()

        edge_dma(0, 0, True)
        edge_dma(0, 0, False)
        gather(0, True)

        @pl.loop(0, (nch0 + 1) // 2)
        def _(g):
            for b in range(2):
                k = g * 2 + b

                @pl.when(k < n_chunks)
                def _():
                    @pl.when(k >= 1)
                    def _():
                        scatter(1 - b, False)

                    @pl.when(k + 1 < n_chunks)
                    def _():
                        edge_dma(k + 1, 1 - b, True)

                    gather(b, False)

                    @pl.when(k + 1 < n_chunks)
                    def _():
                        edge_dma(k + 1, 1 - b, False)
                        gather(1 - b, True)

                    @pl.loop(0, RCH)
                    def _(r):
                        @pl.loop(0, ROW // 16)
                        def _(i):
                            idx = srcb[b, r, pl.ds(i * 16, 16)]
                            dv = plsc.load_gather(dinv_l, [idx])
                            a16[...] = wb[b, r, pl.ds(i * 16, 16)] * dv
                            e0 = r * ROW + i * 16
                            for j in range(16):
                                a = plsc.load_gather(
                                    a16, [jnp.full((16,), j, jnp.int32)])
                                e = e0 + j
                                for f in range(FW):
                                    wv = rowsb[b, e, pl.ds(f * 32, 32)]
                                    lo, hi = _unpack_bf16_pair(wv)
                                    rowsf[b, e, pl.ds(f * 32, 16)] = lo * a
                                    rowsf[b, e, pl.ds(f * 32 + 16, 16)] = \
                                        hi * a

                    scatter(b, True)

        scatter(1, False)
        plsc.subcore_barrier()

        @pl.loop(0, cnt_subs)
        def _(q):
            n0 = node0 + q * SUB
            pltpu.sync_copy(acc_sh.at[pl.ds(n0, SUB)],
                            rowsf.at[0].at[pl.ds(0, SUB)])

            @pl.loop(0, SUB // 16)
            def _(g):
                dvv = dinv_l[pl.ds(n0 + g * 16, 16)]
                for j in range(16):
                    a = dvv[j]
                    e = g * 16 + j
                    for f in range(F3 // 16):
                        sf = pl.ds(f * 16, 16)
                        rowsf[0, e, sf] = rowsf[0, e, sf] * a

            pltpu.sync_copy(rowsf.at[0].at[pl.ds(0, SUB)],
                            acc_out.at[c].at[pl.ds(n0, SUB)])

    return kern(src2d, dst2d, w2d, hb, degp)


def _tc_project(x, wcat_p):
    nb = 5
    bs = N // nb

    def body(x_ref, w_ref, o_ref):
        o_ref[...] = jnp.dot(x_ref[...], w_ref[...],
                             preferred_element_type=jnp.float32
                             ).astype(jnp.bfloat16)

    return pl.pallas_call(
        body,
        grid=(nb,),
        in_specs=[
            pl.BlockSpec((bs, F_IN), lambda i: (i, 0)),
            pl.BlockSpec((F_IN, F3), lambda i: (0, 0)),
        ],
        out_specs=pl.BlockSpec((bs, F3), lambda i: (i, 0)),
        out_shape=jax.ShapeDtypeStruct((N, F3), jnp.bfloat16),
    )(x, wcat_p)


def _tc_gru(acc, hprev, wza, wzb, cz, wra, wrb, cr, wha, whb, ch,
            wlin, blin):
    nb = 5
    bs = N // nb

    def body(a0_ref, a1_ref, hp_ref, wza_ref, wzb_ref, cz_ref,
             wra_ref, wrb_ref, cr_ref, wha_ref, whb_ref, ch_ref,
             wlin_ref, blin_ref, y_ref, hn_ref):
        hp = hp_ref[...]
        agg = a0_ref[0] + a1_ref[0]
        gz = agg[:, :F_OUT]
        gr = agg[:, F_OUT:2 * F_OUT]
        gh = agg[:, 2 * F_OUT:]
        f32 = jnp.float32
        z = jax.nn.sigmoid(jnp.dot(gz, wza_ref[...], preferred_element_type=f32)
                           + jnp.dot(hp, wzb_ref[...], preferred_element_type=f32)
                           + cz_ref[...])
        r = jax.nn.sigmoid(jnp.dot(gr, wra_ref[...], preferred_element_type=f32)
                           + jnp.dot(hp, wrb_ref[...], preferred_element_type=f32)
                           + cr_ref[...])
        ht = jnp.tanh(jnp.dot(gh, wha_ref[...], preferred_element_type=f32)
                      + jnp.dot(hp * r, whb_ref[...], preferred_element_type=f32)
                      + ch_ref[...])
        hn = z * hp + (1.0 - z) * ht
        hn_ref[...] = hn
        y_ref[...] = (jnp.dot(jax.nn.relu(hn), wlin_ref[...],
                              preferred_element_type=f32) + blin_ref[...])

    full = lambda r, c: pl.BlockSpec((r, c), lambda i: (0, 0))
    blk = lambda cdim: pl.BlockSpec((bs, cdim), lambda i: (i, 0))
    acc_blk = lambda ci: pl.BlockSpec((1, bs, F3), lambda i, ci=ci: (ci, i, 0))
    return pl.pallas_call(
        body,
        grid=(nb,),
        in_specs=[
            acc_blk(0), acc_blk(1), blk(F_OUT),
            full(F_OUT, F_OUT), full(F_OUT, F_OUT), full(1, F_OUT),
            full(F_OUT, F_OUT), full(F_OUT, F_OUT), full(1, F_OUT),
            full(F_OUT, F_OUT), full(F_OUT, F_OUT), full(1, F_OUT),
            full(F_OUT, 1), full(1, 1),
        ],
        out_specs=[blk(1), blk(F_OUT)],
        out_shape=[
            jax.ShapeDtypeStruct((N, 1), jnp.float32),
            jax.ShapeDtypeStruct((N, F_OUT), jnp.float32),
        ],
    )(acc, acc, hprev, wza, wzb, cz, wra, wrb, cr,
      wha, whb, ch, wlin, blin)


_PERM = np.empty((F3,), dtype=np.int32)
for _t in range(FW):
    for _j in range(16):
        _PERM[32 * _t + 2 * _j] = 32 * _t + _j
        _PERM[32 * _t + 2 * _j + 1] = 32 * _t + 16 + _j


def kernel(x, edge_index, edge_weight, prev_hidden_state, c,
           Wz_c, bz_c, Wr_c, br_c, Wh_c, bh_c,
           Wz, bz, Wr, br, Wh, bh, Wlin, blin):
    src, dst = edge_index[0], edge_index[1]
    e = src.shape[0]

    grain = NC * NS * DCH * ROW
    epad = -(-e // grain) * grain
    pad = epad - e
    if pad:
        src = jnp.concatenate([src, jnp.zeros((pad,), src.dtype)])
        dst = jnp.concatenate([dst, jnp.zeros((pad,), dst.dtype)])
        edge_weight = jnp.concatenate(
            [edge_weight, jnp.zeros((pad,), edge_weight.dtype)])
    n_rows = epad // ROW
    src2d = src.reshape(n_rows, ROW)
    dst2d = dst.reshape(n_rows, ROW)
    w2d = edge_weight.reshape(n_rows, ROW)

    wcat = jnp.concatenate([Wz_c, Wr_c, Wh_c], axis=1)
    wcat_p = wcat[:, _PERM]
    cz = (bz_c @ Wz[:F_OUT] + bz).reshape(1, F_OUT)
    cr = (br_c @ Wr[:F_OUT] + br).reshape(1, F_OUT)
    ch = (bh_c @ Wh[:F_OUT] + bh).reshape(1, F_OUT)

    degp = _sc_degree(dst2d, w2d, n_rows)
    hb = _tc_project(x, wcat_p)
    acc = _sc_messages(src2d, dst2d, w2d, hb, degp, n_rows)
    y, hn = _tc_gru(acc, prev_hidden_state,
                    Wz[:F_OUT], Wz[F_OUT:], cz,
                    Wr[:F_OUT], Wr[F_OUT:], cr,
                    Wh[:F_OUT], Wh[F_OUT:], ch,
                    Wlin, blin.reshape(1, 1))
    return (y, hn)
